# baseline (speedup 1.0000x reference)
import functools

import jax
import jax.numpy as jnp
from jax import lax
from jax.experimental import pallas as pl
from jax.experimental.pallas import tpu as pltpu
from jax.experimental.pallas import tpu_sc as plsc

_ROW_BLK = 2048
_N_STREAMS = 4


def _fold_one(tbl, w_ref):
    blk, d = tbl.shape
    w_rep = jnp.broadcast_to(w_ref[...], (d, 128))
    yw = jnp.dot(tbl, w_rep, preferred_element_type=jnp.float32)
    yw3 = yw.reshape(blk // 128, 128, 128)
    eye = (
        lax.broadcasted_iota(jnp.int32, (128, 128), 0)
        == lax.broadcasted_iota(jnp.int32, (128, 128), 1)
    )
    sel = jnp.where(eye[None], yw3, 0.0)
    return jnp.sum(sel, axis=1)


def _project_body(*refs):
    t_refs = refs[:_N_STREAMS]
    w_ref, b_ref, out_ref = refs[_N_STREAMS:]
    for q, tq in enumerate(t_refs):
        out_ref[q] = _fold_one(tq[...], w_ref) + b_ref[0, 0]


def _project(table, W, b):
    n, d = table.shape
    stride_blk, grid = _quarter_geometry(n)
    rows_q = grid * _ROW_BLK
    n_pad = _N_STREAMS * rows_q

    def make_spec(q):
        return pl.BlockSpec((_ROW_BLK, d), lambda i, q=q: (i + stride_blk * q, 0))

    t = pl.pallas_call(
        _project_body,
        grid=(grid,),
        in_specs=[make_spec(q) for q in range(_N_STREAMS)]
        + [
            pl.BlockSpec((d, 1), lambda i: (0, 0)),
            pl.BlockSpec((1, 1), lambda i: (0, 0)),
        ],
        out_specs=pl.BlockSpec(
            (_N_STREAMS, _ROW_BLK // 128, 128), lambda i: (0, i, 0)
        ),
        out_shape=jax.ShapeDtypeStruct(
            (_N_STREAMS, rows_q // 128, 128), jnp.float32
        ),
    )(*([table] * _N_STREAMS), W, b.reshape(1, 1))
    return t.reshape(n_pad)


def _quarter_geometry(n):
    stride_blk = n // (_N_STREAMS * _ROW_BLK)
    grid = -(-(n - stride_blk * (_N_STREAMS - 1) * _ROW_BLK) // _ROW_BLK)
    return stride_blk, grid


def _t_position(idx, n):
    stride_blk, grid = _quarter_geometry(n)
    q_rows = stride_blk * _ROW_BLK
    q = jnp.minimum(idx // q_rows, _N_STREAMS - 1)
    return idx + (grid - stride_blk) * _ROW_BLK * q


def _make_pool(nw, nc, ns, rpw, hist, n_t):
    mesh = plsc.VectorSubcoreMesh(core_axis_name="c", subcore_axis_name="s")
    t_slice = n_t // ns

    @functools.partial(
        pl.kernel,
        out_type=jax.ShapeDtypeStruct((nw * rpw,), jnp.float32),
        mesh=mesh,
        scratch_types=[
            pltpu.VMEM((hist * rpw,), jnp.int32),
            pltpu.VMEM((hist * rpw,), jnp.float32),
            pltpu.VMEM((rpw,), jnp.float32),
            pltpu.VMEM_SHARED((n_t,), jnp.float32),
            pltpu.SemaphoreType.DMA,
        ],
    )
    def pool(t_hbm, idx_hbm, out_hbm, idx_v, vals_v, res_v, t_sh, sem):
        cid = lax.axis_index("c")
        sid = lax.axis_index("s")
        wid = sid * nc + cid
        pltpu.sync_copy(
            t_hbm.at[pl.ds(sid * t_slice, t_slice)],
            t_sh.at[pl.ds(sid * t_slice, t_slice)],
        )
        pltpu.sync_copy(idx_hbm.at[wid], idx_v)
        plsc.subcore_barrier()
        pltpu.async_copy(t_sh.at[idx_v], vals_v, sem).wait()
        scale = 1.0 / hist
        for g in range(rpw // 16):
            def body(j, acc, g=g):
                return acc + vals_v[pl.ds(j * rpw + g * 16, 16)]
            acc = lax.fori_loop(0, hist, body, jnp.zeros((16,), jnp.float32))
            res_v[pl.ds(g * 16, 16)] = acc * scale
        pltpu.sync_copy(res_v, out_hbm.at[pl.ds(wid * rpw, rpw)])

    return pool


def kernel(x, table, W, b):
    batch, hist = x.shape
    info = plsc.get_sparse_core_info()
    nc, ns = info.num_cores, info.num_subcores
    nw = nc * ns
    rpw = batch // nw
    t = _project(table, W, b)
    idx = (
        _t_position(x.astype(jnp.int32), table.shape[0])
        .reshape(nw, rpw, hist)
        .transpose(0, 2, 1)
        .reshape(nw, hist * rpw)
    )
    y = _make_pool(nw, nc, ns, rpw, hist, t.shape[0])(t, idx)
    return y.reshape(batch, 1)


# speedup vs baseline: 1.0817x; 1.0817x over previous
import functools

import jax
import jax.numpy as jnp
from jax import lax
from jax.experimental import pallas as pl
from jax.experimental.pallas import tpu as pltpu
from jax.experimental.pallas import tpu_sc as plsc

_ROW_BLK = 4096
_N_STREAMS = 4


def _fold_one(tbl, w_ref):
    blk, d = tbl.shape
    w_rep = jnp.broadcast_to(w_ref[...], (d, 128))
    yw = jnp.dot(tbl, w_rep, preferred_element_type=jnp.float32)
    yw3 = yw.reshape(blk // 128, 128, 128)
    eye = (
        lax.broadcasted_iota(jnp.int32, (128, 128), 0)
        == lax.broadcasted_iota(jnp.int32, (128, 128), 1)
    )
    sel = jnp.where(eye[None], yw3, 0.0)
    return jnp.sum(sel, axis=1)


def _project_body(*refs):
    t_refs = refs[:_N_STREAMS]
    w_ref, b_ref, out_ref = refs[_N_STREAMS:]
    for q, tq in enumerate(t_refs):
        out_ref[q] = _fold_one(tq[...], w_ref) + b_ref[0, 0]


_PIPE_BLK = 8192
_PIPE_DEPTH = 8


def _project_pipe_body(table_any, w_ref, b_ref, out_ref, bufs, sems):
    i = pl.program_id(0)
    n_blocks = pl.num_programs(0)
    n = table_any.shape[0]
    last_start = n - _PIPE_BLK

    def start_copy(blk_idx):
        slot = lax.rem(blk_idx, _PIPE_DEPTH)
        start = jnp.minimum(blk_idx * _PIPE_BLK, last_start)
        pltpu.make_async_copy(
            table_any.at[pl.ds(start, _PIPE_BLK), :],
            bufs.at[slot],
            sems.at[slot],
        ).start()

    @pl.when(i == 0)
    def _prime():
        for k in range(_PIPE_DEPTH):
            start_copy(jnp.int32(k))

    slot = lax.rem(i, _PIPE_DEPTH)
    pltpu.make_async_copy(
        table_any.at[pl.ds(jnp.minimum(i * _PIPE_BLK, last_start), _PIPE_BLK), :],
        bufs.at[slot],
        sems.at[slot],
    ).wait()
    out_ref[...] = _fold_one(bufs[slot], w_ref) + b_ref[0, 0]

    @pl.when(i + _PIPE_DEPTH < n_blocks)
    def _next():
        start_copy(i + _PIPE_DEPTH)


def _project_pipe(table, W, b):
    n, d = table.shape
    grid = (n + _PIPE_BLK - 1) // _PIPE_BLK
    n_pad = grid * _PIPE_BLK
    t = pl.pallas_call(
        _project_pipe_body,
        grid=(grid,),
        in_specs=[
            pl.BlockSpec(memory_space=pltpu.MemorySpace.HBM),
            pl.BlockSpec((d, 1), lambda i: (0, 0)),
            pl.BlockSpec((1, 1), lambda i: (0, 0)),
        ],
        out_specs=pl.BlockSpec((_PIPE_BLK // 128, 128), lambda i: (i, 0)),
        out_shape=jax.ShapeDtypeStruct((n_pad // 128, 128), jnp.float32),
        scratch_shapes=[
            pltpu.VMEM((_PIPE_DEPTH, _PIPE_BLK, d), jnp.float32),
            pltpu.SemaphoreType.DMA((_PIPE_DEPTH,)),
        ],
    )(table, W, b.reshape(1, 1))
    return t.reshape(n_pad)


def _t_position_pipe(idx, n):
    grid = (n + _PIPE_BLK - 1) // _PIPE_BLK
    last_start = n - _PIPE_BLK
    shift = (grid - 1) * _PIPE_BLK - last_start
    return jnp.where(
        idx < (grid - 1) * _PIPE_BLK, idx, idx + shift
    )


def _project(table, W, b):
    n, d = table.shape
    stride_blk, grid = _quarter_geometry(n)
    rows_q = grid * _ROW_BLK
    n_pad = _N_STREAMS * rows_q

    def make_spec(q):
        return pl.BlockSpec((_ROW_BLK, d), lambda i, q=q: (i + stride_blk * q, 0))

    t = pl.pallas_call(
        _project_body,
        grid=(grid,),
        in_specs=[make_spec(q) for q in range(_N_STREAMS)]
        + [
            pl.BlockSpec((d, 1), lambda i: (0, 0)),
            pl.BlockSpec((1, 1), lambda i: (0, 0)),
        ],
        out_specs=pl.BlockSpec(
            (_N_STREAMS, _ROW_BLK // 128, 128), lambda i: (0, i, 0)
        ),
        out_shape=jax.ShapeDtypeStruct(
            (_N_STREAMS, rows_q // 128, 128), jnp.float32
        ),
    )(*([table] * _N_STREAMS), W, b.reshape(1, 1))
    return t.reshape(n_pad)


def _quarter_geometry(n):
    stride_blk = n // (_N_STREAMS * _ROW_BLK)
    grid = -(-(n - stride_blk * (_N_STREAMS - 1) * _ROW_BLK) // _ROW_BLK)
    return stride_blk, grid


def _t_position(idx, n):
    stride_blk, grid = _quarter_geometry(n)
    q_rows = stride_blk * _ROW_BLK
    q = jnp.minimum(idx // q_rows, _N_STREAMS - 1)
    return idx + (grid - stride_blk) * _ROW_BLK * q


def _make_pool(nw, nc, ns, rpw, hist, n_t):
    mesh = plsc.VectorSubcoreMesh(core_axis_name="c", subcore_axis_name="s")
    t_slice = n_t // ns

    @functools.partial(
        pl.kernel,
        out_type=jax.ShapeDtypeStruct((nw * rpw,), jnp.float32),
        mesh=mesh,
        scratch_types=[
            pltpu.VMEM((hist * rpw,), jnp.int32),
            pltpu.VMEM((hist * rpw,), jnp.float32),
            pltpu.VMEM((rpw,), jnp.float32),
            pltpu.VMEM_SHARED((n_t,), jnp.float32),
            pltpu.SemaphoreType.DMA,
        ],
    )
    def pool(t_hbm, idx_hbm, out_hbm, idx_v, vals_v, res_v, t_sh, sem):
        cid = lax.axis_index("c")
        sid = lax.axis_index("s")
        wid = sid * nc + cid
        pltpu.sync_copy(
            t_hbm.at[pl.ds(sid * t_slice, t_slice)],
            t_sh.at[pl.ds(sid * t_slice, t_slice)],
        )
        pltpu.sync_copy(idx_hbm.at[wid], idx_v)
        plsc.subcore_barrier()
        pltpu.async_copy(t_sh.at[idx_v], vals_v, sem).wait()
        scale = 1.0 / hist
        for g in range(rpw // 16):
            def body(j, acc, g=g):
                return acc + vals_v[pl.ds(j * rpw + g * 16, 16)]
            acc = lax.fori_loop(0, hist, body, jnp.zeros((16,), jnp.float32))
            res_v[pl.ds(g * 16, 16)] = acc * scale
        pltpu.sync_copy(res_v, out_hbm.at[pl.ds(wid * rpw, rpw)])

    return pool


def kernel(x, table, W, b):
    batch, hist = x.shape
    info = plsc.get_sparse_core_info()
    nc, ns = info.num_cores, info.num_subcores
    nw = nc * ns
    rpw = batch // nw
    t = _project_pipe(table, W, b)
    idx = (
        _t_position_pipe(x.astype(jnp.int32), table.shape[0])
        .reshape(nw, rpw, hist)
        .transpose(0, 2, 1)
        .reshape(nw, hist * rpw)
    )
    y = _make_pool(nw, nc, ns, rpw, hist, t.shape[0])(t, idx)
    return y.reshape(batch, 1)


# speedup vs baseline: 1.0891x; 1.0068x over previous
import functools

import jax
import jax.numpy as jnp
from jax import lax
from jax.experimental import pallas as pl
from jax.experimental.pallas import tpu as pltpu
from jax.experimental.pallas import tpu_sc as plsc

_ROW_BLK = 4096
_N_STREAMS = 4


def _fold_one(tbl, w_ref):
    blk, d = tbl.shape
    w_rep = jnp.broadcast_to(w_ref[...], (d, 128))
    yw = jnp.dot(tbl, w_rep, preferred_element_type=jnp.float32)
    yw3 = yw.reshape(blk // 128, 128, 128)
    eye = (
        lax.broadcasted_iota(jnp.int32, (128, 128), 0)
        == lax.broadcasted_iota(jnp.int32, (128, 128), 1)
    )
    sel = jnp.where(eye[None], yw3, 0.0)
    return jnp.sum(sel, axis=1)


def _project_body(*refs):
    t_refs = refs[:_N_STREAMS]
    w_ref, b_ref, out_ref = refs[_N_STREAMS:]
    for q, tq in enumerate(t_refs):
        out_ref[q] = _fold_one(tq[...], w_ref) + b_ref[0, 0]


def _project(table, W, b):
    n, d = table.shape
    stride_blk, grid = _quarter_geometry(n)
    rows_q = grid * _ROW_BLK
    n_pad = _N_STREAMS * rows_q

    def make_spec(q):
        return pl.BlockSpec((_ROW_BLK, d), lambda i, q=q: (i + stride_blk * q, 0))

    t = pl.pallas_call(
        _project_body,
        grid=(grid,),
        in_specs=[make_spec(q) for q in range(_N_STREAMS)]
        + [
            pl.BlockSpec((d, 1), lambda i: (0, 0)),
            pl.BlockSpec((1, 1), lambda i: (0, 0)),
        ],
        out_specs=pl.BlockSpec(
            (_N_STREAMS, _ROW_BLK // 128, 128), lambda i: (0, i, 0)
        ),
        out_shape=jax.ShapeDtypeStruct(
            (_N_STREAMS, rows_q // 128, 128), jnp.float32
        ),
    )(*([table] * _N_STREAMS), W, b.reshape(1, 1))
    return t.reshape(n_pad)


def _quarter_geometry(n):
    stride_blk = n // (_N_STREAMS * _ROW_BLK)
    grid = -(-(n - stride_blk * (_N_STREAMS - 1) * _ROW_BLK) // _ROW_BLK)
    return stride_blk, grid


def _t_position(idx, n):
    stride_blk, grid = _quarter_geometry(n)
    q_rows = stride_blk * _ROW_BLK
    q = jnp.minimum(idx // q_rows, _N_STREAMS - 1)
    return idx + (grid - stride_blk) * _ROW_BLK * q


def _make_pool(nw, nc, ns, rpw, hist, n_t):
    mesh = plsc.VectorSubcoreMesh(core_axis_name="c", subcore_axis_name="s")
    t_slice = n_t // ns

    @functools.partial(
        pl.kernel,
        out_type=jax.ShapeDtypeStruct((nw * rpw,), jnp.float32),
        mesh=mesh,
        scratch_types=[
            pltpu.VMEM((hist * rpw,), jnp.int32),
            pltpu.VMEM((hist * rpw,), jnp.float32),
            pltpu.VMEM((rpw,), jnp.float32),
            pltpu.VMEM_SHARED((n_t,), jnp.float32),
            pltpu.SemaphoreType.DMA,
        ],
    )
    def pool(t_hbm, idx_hbm, out_hbm, idx_v, vals_v, res_v, t_sh, sem):
        cid = lax.axis_index("c")
        sid = lax.axis_index("s")
        wid = sid * nc + cid
        pltpu.sync_copy(
            t_hbm.at[pl.ds(sid * t_slice, t_slice)],
            t_sh.at[pl.ds(sid * t_slice, t_slice)],
        )
        pltpu.sync_copy(idx_hbm.at[wid], idx_v)
        plsc.subcore_barrier()
        pltpu.async_copy(t_sh.at[idx_v], vals_v, sem).wait()
        scale = 1.0 / hist
        for g in range(rpw // 16):
            def body(j, acc, g=g):
                return acc + vals_v[pl.ds(j * rpw + g * 16, 16)]
            acc = lax.fori_loop(
                0, hist, body, jnp.zeros((16,), jnp.float32), unroll=4
            )
            res_v[pl.ds(g * 16, 16)] = acc * scale
        pltpu.sync_copy(res_v, out_hbm.at[pl.ds(wid * rpw, rpw)])

    return pool


def kernel(x, table, W, b):
    batch, hist = x.shape
    info = plsc.get_sparse_core_info()
    nc, ns = info.num_cores, info.num_subcores
    nw = nc * ns
    rpw = batch // nw
    t = _project(table, W, b)
    idx = (
        _t_position(x.astype(jnp.int32), table.shape[0])
        .reshape(nw, rpw, hist)
        .transpose(0, 2, 1)
        .reshape(nw, hist * rpw)
    )
    y = _make_pool(nw, nc, ns, rpw, hist, t.shape[0])(t, idx)
    return y.reshape(batch, 1)
